# baseline (speedup 1.0000x reference)
import functools

import jax
import jax.numpy as jnp
from jax import lax
from jax.experimental import pallas as pl
from jax.experimental.pallas import tpu as pltpu
from jax.experimental.pallas import tpu_sc as plsc

_BATCH = 4096
_VOCAB = 100000
_DIM = 64
_NC = 2
_NS = 16
_NW = _NC * _NS
_RPW = _DIM // _NW
_LANES = 16
_GROUPS = _BATCH // _LANES
_C0 = 43520
_C1 = 43520
_C2 = _VOCAB - _C0 - _C1
_OFFS = (0, _C0, _C0 + _C1)
_SIZES = (_C0, _C1, _C2)
_LCAP = _BATCH + _LANES
_POSBITS = 13
_POSMASK = (1 << _POSBITS) - 1


@functools.partial(
    pl.kernel,
    mesh=plsc.VectorSubcoreMesh(core_axis_name="c", subcore_axis_name="s"),
    out_type=jax.ShapeDtypeStruct((_DIM, _BATCH), jnp.float32),
    scratch_types=[
        pltpu.VMEM((_BATCH,), jnp.int32),
        pltpu.VMEM((_C0,), jnp.float32),
        pltpu.VMEM((_C1,), jnp.float32),
        pltpu.VMEM((_C2,), jnp.float32),
        pltpu.VMEM((_LCAP,), jnp.int32),
        pltpu.VMEM((_LCAP,), jnp.int32),
        pltpu.VMEM((_LCAP,), jnp.int32),
        pltpu.VMEM((_LCAP,), jnp.float32),
        pltpu.VMEM((_LCAP,), jnp.float32),
        pltpu.SemaphoreType.DMA,
        pltpu.SemaphoreType.DMA,
        pltpu.SemaphoreType.DMA,
    ],
    compiler_params=pltpu.CompilerParams(needs_layout_passes=False),
)
def _gather_add(xt_hbm, tablet_hbm, idx_hbm, outt_hbm,
                idx_v, buf0, buf1, buf2, list0, list1, list2,
                x_v, o_v, sem0, sem1, sem2):
    wid = lax.axis_index("s") * _NC + lax.axis_index("c")
    c0row = wid * _RPW
    bufs = (buf0, buf1, buf2)
    lists = (list0, list1, list2)
    sems = (sem0, sem1, sem2)

    def issue(row, j):
        return pltpu.async_copy(
            tablet_hbm.at[row, pl.ds(_OFFS[j], _SIZES[j])], bufs[j], sems[j])

    cps = [issue(c0row, j) for j in range(3)]
    pltpu.sync_copy(idx_hbm, idx_v)

    lane = lax.iota(jnp.int32, _LANES)

    def part_body(g, ns):
        n0, n1, n2 = ns
        sl = pl.ds(g * _LANES, _LANES)
        iv = idx_v[sl]
        key = (iv << _POSBITS) | (g * _LANES + lane)
        m0 = iv < _OFFS[1]
        m2 = iv >= _OFFS[2]
        m1 = jnp.logical_not(jnp.logical_or(m0, m2))
        plsc.store_compressed(list0.at[pl.ds(n0, _LANES)], key, mask=m0)
        plsc.store_compressed(list1.at[pl.ds(n1, _LANES)], key, mask=m1)
        plsc.store_compressed(list2.at[pl.ds(n2, _LANES)], key, mask=m2)
        return (n0 + jnp.sum(m0.astype(jnp.int32)),
                n1 + jnp.sum(m1.astype(jnp.int32)),
                n2 + jnp.sum(m2.astype(jnp.int32)))

    zero = jnp.int32(0)
    n0, n1, n2 = lax.fori_loop(0, _GROUPS, part_body, (zero, zero, zero))
    ns = (n0, n1, n2)
    for j in range(3):
        pad = jnp.full((_LANES,), (_OFFS[j] << _POSBITS) | _BATCH, jnp.int32)
        lists[j][pl.ds(ns[j], _LANES)] = pad

    def make_pass(j, lst):
        def pass_body(g, carry):
            sl = pl.ds(g * _LANES, _LANES)
            k = lst[sl]
            p = k & _POSMASK
            i = lax.shift_right_logical(k, _POSBITS) - _OFFS[j]
            v = plsc.load_gather(bufs[j], [i])
            xv = plsc.load_gather(x_v, [p])
            plsc.store_scatter(o_v, [p], xv + v)
            return carry
        return pass_body

    for t in range(_RPW):
        row = c0row + t
        pltpu.sync_copy(xt_hbm.at[row], x_v.at[pl.ds(0, _BATCH)])
        for j in range(3):
            cps[j].wait()
            gmax = (ns[j] + _LANES - 1) // _LANES
            lax.fori_loop(0, gmax, make_pass(j, lists[j]), 0)
            if t + 1 < _RPW:
                cps[j] = issue(row + 1, j)
        pltpu.sync_copy(o_v.at[pl.ds(0, _BATCH)], outt_hbm.at[row])


def kernel(x, const, indices):
    out_t = _gather_add(x.T, const.T, indices.astype(jnp.int32))
    return out_t.T
